# baseline (speedup 1.0000x reference)
import functools

import jax
import jax.numpy as jnp
from jax import lax
from jax.experimental import pallas as pl
from jax.experimental.pallas import tpu as pltpu
from jax.experimental.pallas import tpu_sc as plsc

N_NODES = 10000
N_TILES = 16
ROWS_T = 640
NPAD = N_TILES * ROWS_T
DH = 128
CH = 128
NHALF = 2


def _agg_factory(nchunks: int, compute_deg: bool):
  assert nchunks % (2 * NHALF) == 0 and nchunks >= 4 * NHALF
  nloc = nchunks // NHALF
  mesh = plsc.VectorSubcoreMesh(core_axis_name="c", subcore_axis_name="s")
  out_type = [
      jax.ShapeDtypeStruct((NPAD, DH), jnp.float32),
      jax.ShapeDtypeStruct((NPAD, DH), jnp.float32),
  ]
  if compute_deg:
    out_type.append(jax.ShapeDtypeStruct((NPAD,), jnp.float32))
  scratch = [
      pltpu.VMEM_SHARED((NPAD, DH), jnp.float32),
      pltpu.VMEM_SHARED((NPAD,), jnp.float32),
      pltpu.VMEM((nloc, CH), jnp.int32),
      pltpu.VMEM((nloc, CH), jnp.int32),
      [pltpu.VMEM((CH, DH), jnp.float32)] * 2,
      pltpu.VMEM((CH,), jnp.float32),
      [pltpu.SemaphoreType.DMA] * 2,
      [pltpu.SemaphoreType.DMA] * 2,
  ]

  def body(x0, x1, src4, dst4, z2, z1, *rest):
    if compute_deg:
      a0_out, a1_out, deg_out = rest[:3]
      acc, dacc, srcs, dsts, rows, ones, gsem, ssem = rest[3:]
    else:
      a0_out, a1_out = rest[:2]
      deg_out = None
      acc, dacc, srcs, dsts, rows, ones, gsem, ssem = rest[2:]
    c = lax.axis_index("c")
    t = lax.axis_index("s")
    base = t * ROWS_T

    pltpu.sync_copy(z2, acc.at[pl.ds(base, ROWS_T)])
    if compute_deg:
      @pl.when(c == 1)
      def _():
        pltpu.sync_copy(z1, dacc.at[pl.ds(base, ROWS_T)])
      for k in range(CH // 16):
        ones[pl.ds(k * 16, 16)] = jnp.full((16,), 1.0, jnp.float32)

    def pipeline(xh, do_deg):
      def gather(j, s):
        pltpu.async_copy(xh.at[srcs.at[j]], rows[s], gsem[s])

      def wait_gather(s):
        pltpu.make_async_copy(xh.at[srcs.at[0]], rows[s], gsem[s]).wait()

      def scatter(j, s):
        pltpu.async_copy(rows[s], acc.at[dsts.at[j]], ssem[s], add=True)
        if do_deg:
          pltpu.async_copy(ones, dacc.at[dsts.at[j]], ssem[s], add=True)

      def wait_scatter(s):
        pltpu.make_async_copy(rows[s], acc.at[dsts.at[0]], ssem[s]).wait()
        if do_deg:
          pltpu.make_async_copy(ones, dacc.at[dsts.at[0]], ssem[s]).wait()

      def step(j, s, do_ws, do_refill):
        wait_gather(s)
        scatter(j, s)
        if do_refill:
          if do_ws:
            wait_scatter(1 - s)
          gather(j + 1, 1 - s)

      for h in range(NHALF):
        pltpu.sync_copy(src4.at[t, h], srcs)
        pltpu.sync_copy(dst4.at[t, h], dsts)
        gather(0, 0)
        step(0, 0, do_ws=False, do_refill=True)
        step(1, 1, do_ws=True, do_refill=True)

        def pair(g, carry):
          j0 = g * 2
          step(j0, 0, do_ws=True, do_refill=True)
          step(j0 + 1, 1, do_ws=True, do_refill=True)
          return carry
        lax.fori_loop(1, nloc // 2 - 1, pair, 0)

        step(nloc - 2, 0, do_ws=True, do_refill=True)
        step(nloc - 1, 1, do_ws=False, do_refill=False)
        wait_scatter(0)
        wait_scatter(1)

    plsc.subcore_barrier()

    @pl.when(c == 0)
    def _():
      pipeline(x0, False)

    @pl.when(c == 1)
    def _():
      pipeline(x1, compute_deg)

    plsc.subcore_barrier()

    @pl.when(c == 0)
    def _():
      pltpu.sync_copy(acc.at[pl.ds(base, ROWS_T)], a0_out.at[pl.ds(base, ROWS_T)])

    @pl.when(c == 1)
    def _():
      pltpu.sync_copy(acc.at[pl.ds(base, ROWS_T)], a1_out.at[pl.ds(base, ROWS_T)])
      if compute_deg:
        pltpu.sync_copy(dacc.at[pl.ds(base, ROWS_T)],
                        deg_out.at[pl.ds(base, ROWS_T)])

  return pl.kernel(body, out_type=tuple(out_type), mesh=mesh,
                   scratch_types=scratch)


BR = 400


def _tc1_body(a0, a1, dg, x, wl1, bl1, wr1, wl2, wr2, bl2, q0, q1, r_out):
  r = 1.0 / jnp.maximum(dg[...], 1.0)
  a = jnp.concatenate([a0[...], a1[...]], axis=1) * r
  h = (jnp.dot(a, wl1[...], preferred_element_type=jnp.float32)
       + jnp.dot(x[...], wr1[...], preferred_element_type=jnp.float32)
       + bl1[...])
  h = jnp.maximum(h, 0.0)
  q = jnp.dot(h, wl2[...], preferred_element_type=jnp.float32)
  q0[...] = q[:, :DH]
  q1[...] = q[:, DH:]
  r_out[...] = (jnp.dot(h, wr2[...], preferred_element_type=jnp.float32)
                + bl2[...])


def _tc2_body(s0, s1, dg, r_in, o):
  r = 1.0 / jnp.maximum(dg[...], 1.0)
  s = jnp.concatenate([s0[...], s1[...]], axis=1) * r
  o[...] = jnp.maximum(s + r_in[...], 0.0)


@functools.cache
def _tc1(d_in, d_hid, d_out):
  row_spec = lambda w: pl.BlockSpec((BR, w), lambda i: (i, 0))
  full = lambda s: pl.BlockSpec(s, lambda i: (0, 0))
  return pl.pallas_call(
      _tc1_body,
      grid=(N_NODES // BR,),
      in_specs=[
          row_spec(DH), row_spec(DH), row_spec(1), row_spec(d_in),
          full((d_in, d_hid)), full((1, d_hid)), full((d_in, d_hid)),
          full((d_hid, d_out)), full((d_hid, d_out)), full((1, d_out)),
      ],
      out_specs=[row_spec(DH), row_spec(DH), row_spec(d_out)],
      out_shape=[
          jax.ShapeDtypeStruct((N_NODES, DH), jnp.float32),
          jax.ShapeDtypeStruct((N_NODES, DH), jnp.float32),
          jax.ShapeDtypeStruct((N_NODES, d_out), jnp.float32),
      ],
  )


@functools.cache
def _tc2(d_out):
  row_spec = lambda w: pl.BlockSpec((BR, w), lambda i: (i, 0))
  return pl.pallas_call(
      _tc2_body,
      grid=(N_NODES // BR,),
      in_specs=[row_spec(DH), row_spec(DH), row_spec(1), row_spec(d_out)],
      out_specs=row_spec(d_out),
      out_shape=jax.ShapeDtypeStruct((N_NODES, d_out), jnp.float32),
  )


@functools.cache
def _agg(nchunks, compute_deg):
  return _agg_factory(nchunks, compute_deg)


def kernel(x, edge_index, Wl1, bl1, Wr1, Wl2, bl2, Wr2):
  src = edge_index[0].astype(jnp.int32)
  dst = edge_index[1].astype(jnp.int32)
  e = src.shape[0]
  ept = e // N_TILES
  nch = (ept + CH - 1) // CH
  nch = -(-nch // (2 * NHALF)) * (2 * NHALF)
  pad = nch * CH - ept
  src3 = jnp.concatenate(
      [src.reshape(N_TILES, ept),
       jnp.zeros((N_TILES, pad), jnp.int32)],
      axis=1).reshape(N_TILES, NHALF, nch // NHALF, CH)
  dst3 = jnp.concatenate(
      [dst.reshape(N_TILES, ept),
       jnp.full((N_TILES, pad), N_NODES, jnp.int32)],
      axis=1).reshape(N_TILES, NHALF, nch // NHALF, CH)
  z2 = jnp.zeros((ROWS_T, DH), jnp.float32)
  z1 = jnp.zeros((ROWS_T,), jnp.float32)

  a0, a1, degp = _agg(nch, True)(x[:, :DH], x[:, DH:], src3, dst3, z2, z1)
  deg2 = degp.reshape(NPAD, 1)

  d_in, d_hid = Wl1.shape
  d_out = Wl2.shape[1]
  q0, q1, r_mat = _tc1(d_in, d_hid, d_out)(
      a0, a1, deg2, x, Wl1, bl1.reshape(1, -1), Wr1, Wl2, Wr2,
      bl2.reshape(1, -1))

  s0, s1 = _agg(nch, False)(q0, q1, src3, dst3, z2, z1)
  return _tc2(d_out)(s0, s1, deg2, r_mat)


# speedup vs baseline: 1.1107x; 1.1107x over previous
import functools

import jax
import jax.numpy as jnp
from jax import lax
from jax.experimental import pallas as pl
from jax.experimental.pallas import tpu as pltpu
from jax.experimental.pallas import tpu_sc as plsc

N_NODES = 10000
N_TILES = 16
ROWS_T = 640
NPAD = N_TILES * ROWS_T
DH = 128
CH = 96
NSLOT = 2


def _agg_factory(nchunks: int, compute_deg: bool):
  assert nchunks % NSLOT == 0 and nchunks // NSLOT >= 3
  ngroups = nchunks // NSLOT
  mesh = plsc.VectorSubcoreMesh(core_axis_name="c", subcore_axis_name="s")
  out_type = [
      jax.ShapeDtypeStruct((NPAD, DH), jnp.float32),
      jax.ShapeDtypeStruct((NPAD, DH), jnp.float32),
  ]
  if compute_deg:
    out_type.append(jax.ShapeDtypeStruct((NPAD,), jnp.float32))
  scratch = [
      pltpu.VMEM_SHARED((NPAD, DH), jnp.float32),
      pltpu.VMEM_SHARED((NPAD,), jnp.float32),
      pltpu.VMEM((nchunks, CH), jnp.int32),
      pltpu.VMEM((nchunks, CH), jnp.int32),
      [pltpu.SemaphoreType.DMA] * NSLOT,
      [pltpu.SemaphoreType.DMA] * NSLOT,
  ]

  def body(x0, x1, src3, dst3, *rest):
    if compute_deg:
      a0_out, a1_out, deg_out = rest[:3]
      acc, dacc, srcs, dsts, gsem, ssem = rest[3:]
    else:
      a0_out, a1_out = rest[:2]
      deg_out = None
      acc, dacc, srcs, dsts, gsem, ssem = rest[2:]
    c = lax.axis_index("c")
    t = lax.axis_index("s")
    base = t * ROWS_T

    def scoped(rows, ones, zvec):
      zero16 = jnp.zeros((16,), jnp.float32)

      def zrow(i, carry):
        for k in range(DH // 16):
          rows[0][i, pl.ds(k * 16, 16)] = zero16
        return carry
      lax.fori_loop(0, CH, zrow, 0)
      for m in range(ROWS_T // CH):
        pltpu.sync_copy(rows[0], acc.at[pl.ds(base + m * CH, CH)])
      rem = ROWS_T % CH
      if rem:
        pltpu.sync_copy(rows[0].at[pl.ds(0, rem)],
                        acc.at[pl.ds(base + ROWS_T - rem, rem)])
      if compute_deg:
        def zvrow(i, carry):
          zvec[pl.ds(i * 16, 16)] = zero16
          return carry
        lax.fori_loop(0, ROWS_T // 16, zvrow, 0)
        @pl.when(c == 1)
        def _():
          pltpu.sync_copy(zvec, dacc.at[pl.ds(base, ROWS_T)])
      pltpu.sync_copy(src3.at[t], srcs)
      pltpu.sync_copy(dst3.at[t], dsts)
      if compute_deg:
        for k in range(CH // 16):
          ones[pl.ds(k * 16, 16)] = jnp.full((16,), 1.0, jnp.float32)

      def pipeline(xh, do_deg):
        def gather(j, s):
          pltpu.async_copy(xh.at[srcs.at[j]], rows[s], gsem[s])

        def wait_gather(s):
          pltpu.make_async_copy(xh.at[srcs.at[0]], rows[s], gsem[s]).wait()

        def scatter(j, s):
          pltpu.async_copy(rows[s], acc.at[dsts.at[j]], ssem[s], add=True)
          if do_deg:
            pltpu.async_copy(ones, dacc.at[dsts.at[j]], ssem[s], add=True)

        def wait_scatter(s):
          pltpu.make_async_copy(rows[s], acc.at[dsts.at[0]], ssem[s]).wait()
          if do_deg:
            pltpu.make_async_copy(ones, dacc.at[dsts.at[0]], ssem[s]).wait()

        def step(j, s, do_ws, do_refill):
          wait_gather(s)
          scatter(j, s)
          if do_refill:
            s1 = (s + 1) % NSLOT
            if do_ws:
              wait_scatter(s1)
            gather(j + 1, s1)

        gather(0, 0)
        for s in range(NSLOT):
          step(s, s, do_ws=(s == NSLOT - 1), do_refill=True)

        def grp(g, carry):
          j0 = g * NSLOT
          for s in range(NSLOT):
            step(j0 + s, s, do_ws=True, do_refill=True)
          return carry
        lax.fori_loop(1, ngroups - 1, grp, 0)

        j0 = (ngroups - 1) * NSLOT
        for s in range(NSLOT):
          step(j0 + s, s, do_ws=True, do_refill=(s < NSLOT - 1))
        for s in range(NSLOT):
          wait_scatter(s)

      plsc.subcore_barrier()

      @pl.when(c == 0)
      def _():
        pipeline(x0, False)

      @pl.when(c == 1)
      def _():
        pipeline(x1, compute_deg)

    pl.run_scoped(
        scoped,
        [pltpu.VMEM((CH, DH), jnp.float32)] * NSLOT,
        pltpu.VMEM((CH,), jnp.float32),
        pltpu.VMEM((ROWS_T,), jnp.float32),
    )

    plsc.subcore_barrier()

    @pl.when(c == 0)
    def _():
      pltpu.sync_copy(acc.at[pl.ds(base, ROWS_T)], a0_out.at[pl.ds(base, ROWS_T)])

    @pl.when(c == 1)
    def _():
      pltpu.sync_copy(acc.at[pl.ds(base, ROWS_T)], a1_out.at[pl.ds(base, ROWS_T)])
      if compute_deg:
        pltpu.sync_copy(dacc.at[pl.ds(base, ROWS_T)],
                        deg_out.at[pl.ds(base, ROWS_T)])

  return pl.kernel(body, out_type=tuple(out_type), mesh=mesh,
                   scratch_types=scratch,
                   compiler_params=pltpu.CompilerParams(
                       use_tc_tiling_on_sc=False))


BR = 400


def _tc1_body(a0, a1, dg, x, wl1, bl1, wr1, wl2, wr2, bl2, q0, q1, r_out):
  r = 1.0 / jnp.maximum(dg[...], 1.0)
  a = jnp.concatenate([a0[...], a1[...]], axis=1) * r
  h = (jnp.dot(a, wl1[...], preferred_element_type=jnp.float32)
       + jnp.dot(x[...], wr1[...], preferred_element_type=jnp.float32)
       + bl1[...])
  h = jnp.maximum(h, 0.0)
  q = jnp.dot(h, wl2[...], preferred_element_type=jnp.float32)
  q0[...] = q[:, :DH]
  q1[...] = q[:, DH:]
  r_out[...] = (jnp.dot(h, wr2[...], preferred_element_type=jnp.float32)
                + bl2[...])


def _tc2_body(s0, s1, dg, r_in, o):
  r = 1.0 / jnp.maximum(dg[...], 1.0)
  s = jnp.concatenate([s0[...], s1[...]], axis=1) * r
  o[...] = jnp.maximum(s + r_in[...], 0.0)


@functools.cache
def _tc1(d_in, d_hid, d_out):
  row_spec = lambda w: pl.BlockSpec((BR, w), lambda i: (i, 0))
  full = lambda s: pl.BlockSpec(s, lambda i: (0, 0))
  return pl.pallas_call(
      _tc1_body,
      grid=(N_NODES // BR,),
      in_specs=[
          row_spec(DH), row_spec(DH), row_spec(1), row_spec(d_in),
          full((d_in, d_hid)), full((1, d_hid)), full((d_in, d_hid)),
          full((d_hid, d_out)), full((d_hid, d_out)), full((1, d_out)),
      ],
      out_specs=[row_spec(DH), row_spec(DH), row_spec(d_out)],
      out_shape=[
          jax.ShapeDtypeStruct((N_NODES, DH), jnp.float32),
          jax.ShapeDtypeStruct((N_NODES, DH), jnp.float32),
          jax.ShapeDtypeStruct((N_NODES, d_out), jnp.float32),
      ],
  )


@functools.cache
def _tc2(d_out):
  row_spec = lambda w: pl.BlockSpec((BR, w), lambda i: (i, 0))
  return pl.pallas_call(
      _tc2_body,
      grid=(N_NODES // BR,),
      in_specs=[row_spec(DH), row_spec(DH), row_spec(1), row_spec(d_out)],
      out_specs=row_spec(d_out),
      out_shape=jax.ShapeDtypeStruct((N_NODES, d_out), jnp.float32),
  )


@functools.cache
def _agg(nchunks, compute_deg):
  return _agg_factory(nchunks, compute_deg)


def kernel(x, edge_index, Wl1, bl1, Wr1, Wl2, bl2, Wr2):
  src = edge_index[0].astype(jnp.int32)
  dst = edge_index[1].astype(jnp.int32)
  e = src.shape[0]
  ept = e // N_TILES
  nch = (ept + CH - 1) // CH
  nch = -(-nch // NSLOT) * NSLOT
  pad = nch * CH - ept
  src3 = jnp.concatenate(
      [src.reshape(N_TILES, ept),
       jnp.zeros((N_TILES, pad), jnp.int32)],
      axis=1).reshape(N_TILES, nch, CH)
  dst3 = jnp.concatenate(
      [dst.reshape(N_TILES, ept),
       jnp.full((N_TILES, pad), N_NODES, jnp.int32)],
      axis=1).reshape(N_TILES, nch, CH)
  a0, a1, degp = _agg(nch, True)(x[:, :DH], x[:, DH:], src3, dst3)
  deg2 = degp.reshape(NPAD, 1)

  d_in, d_hid = Wl1.shape
  d_out = Wl2.shape[1]
  q0, q1, r_mat = _tc1(d_in, d_hid, d_out)(
      a0, a1, deg2, x, Wl1, bl1.reshape(1, -1), Wr1, Wl2, Wr2,
      bl2.reshape(1, -1))

  s0, s1 = _agg(nch, False)(q0, q1, src3, dst3)
  return _tc2(d_out)(s0, s1, deg2, r_mat)
